# Initial kernel scaffold; baseline (speedup 1.0000x reference)
import jax
import jax.numpy as jnp
from jax.experimental import pallas as pl


def kernel(drug_features, protein_features, cell_line_features, disease_features, pp_edge_index, clp_src, clp_dst, W_drug, b_drug, W_prot, b_prot, W_cell, b_cell, W_dis, b_dis, W_self1, W_neigh1, b_sage1, W_self2, W_neigh2, b_sage2, W_gcn, b_gcn):
    raise NotImplementedError("write your pallas kernel here")


# speedup vs baseline: 4.3018x; 4.3018x over previous
import functools

import jax
import jax.numpy as jnp
from jax import lax
from jax.experimental import pallas as pl
from jax.experimental.pallas import tpu as pltpu
from jax.experimental.pallas import tpu_sc as plsc

N_PROT = 10000
N_CELL = 1024
H = 128

NC, NS = 2, 16
NW = NC * NS
CHUNK = 128

BIN_PROT = 10240
BIN_CELL = 2048


def _pad_len(e, per):
    return ((e + per - 1) // per) * per


def _make_degrees(n_pp_chunks, n_clp_chunks):
    mesh = plsc.VectorSubcoreMesh(
        core_axis_name="c", subcore_axis_name="s", num_cores=NC,
        num_subcores=NS)
    pp_per = BIN_PROT // NS
    cs_per = BIN_CELL // NS

    @functools.partial(
        pl.kernel,
        out_type=(
            jax.ShapeDtypeStruct((NC, BIN_PROT, H), jnp.float32),
            jax.ShapeDtypeStruct((NC, BIN_CELL, H), jnp.float32),
            jax.ShapeDtypeStruct((NC, BIN_PROT, H), jnp.float32),
        ),
        mesh=mesh,
        scratch_types=[
            pltpu.VMEM((CHUNK,), jnp.int32),
            pltpu.VMEM((CHUNK, H), jnp.float32),
            pltpu.VMEM((CHUNK, H), jnp.float32),
            pltpu.VMEM_SHARED((BIN_PROT, H), jnp.float32),
        ],
    )
    def deg(ppd_hbm, cls_hbm, cld_hbm, out_pp, out_cs, out_cd,
            idxb, onesb, zrows, acc):
        cid = lax.axis_index("c")
        sid = lax.axis_index("s")
        wid = cid * NS + sid

        ov = jnp.ones((16,), jnp.float32)
        zv = jnp.zeros((16,), jnp.float32)

        def fill(r, carry):
            for c in range(8):
                onesb[r, pl.ds(c * 16, 16)] = ov
                zrows[r, pl.ds(c * 16, 16)] = zv
            return carry

        lax.fori_loop(0, CHUNK, fill, 0)

        def zero_big():
            base_r = sid * pp_per
            for k in range(pp_per // CHUNK):
                pltpu.sync_copy(zrows,
                                acc.at[pl.ds(base_r + k * CHUNK, CHUNK)])

        zero_big()
        plsc.subcore_barrier()

        def run(idx_hbm, a, n_chunks):
            def body(j, carry):
                base = (wid * n_chunks + j) * CHUNK
                pltpu.sync_copy(idx_hbm.at[pl.ds(base, CHUNK)], idxb)
                pltpu.sync_copy(onesb, a.at[idxb], add=True)
                return carry
            lax.fori_loop(0, n_chunks, body, 0)

        base_r = sid * pp_per

        run(ppd_hbm, acc, n_pp_chunks)
        plsc.subcore_barrier()
        pltpu.sync_copy(acc.at[pl.ds(base_r, pp_per)],
                        out_pp.at[cid, pl.ds(base_r, pp_per)])
        plsc.subcore_barrier()

        zero_big()
        plsc.subcore_barrier()
        run(cld_hbm, acc, n_clp_chunks)
        plsc.subcore_barrier()
        pltpu.sync_copy(acc.at[pl.ds(base_r, pp_per)],
                        out_cd.at[cid, pl.ds(base_r, pp_per)])
        plsc.subcore_barrier()

        zero_big()
        plsc.subcore_barrier()
        run(cls_hbm, acc, n_clp_chunks)
        plsc.subcore_barrier()
        pltpu.sync_copy(acc.at[pl.ds(sid * cs_per, cs_per)],
                        out_cs.at[cid, pl.ds(sid * cs_per, cs_per)])

    return deg


def _make_segsum(n_chunks, n_bins):
    mesh = plsc.VectorSubcoreMesh(
        core_axis_name="c", subcore_axis_name="s", num_cores=NC,
        num_subcores=NS)
    rows_per_sub = n_bins // NS

    @functools.partial(
        pl.kernel,
        out_type=jax.ShapeDtypeStruct((NC, n_bins, H), jnp.float32),
        mesh=mesh,
        scratch_types=[
            pltpu.VMEM((2, CHUNK), jnp.int32),
            pltpu.VMEM((2, CHUNK), jnp.int32),
            pltpu.VMEM((2, CHUNK, H), jnp.float32),
            pltpu.VMEM_SHARED((n_bins, H), jnp.float32),
            pltpu.SemaphoreType.DMA,
            pltpu.SemaphoreType.DMA,
        ],
    )
    def seg(src_hbm, dst_hbm, table_hbm, out_hbm,
            sidx, didx, rows, acc, sem0, sem1):
        cid = lax.axis_index("c")
        sid = lax.axis_index("s")
        wid = cid * NS + sid

        zv = jnp.zeros((16,), jnp.float32)

        def zbody(r, carry):
            for c in range(8):
                rows[0, r, pl.ds(c * 16, 16)] = zv
            return carry

        lax.fori_loop(0, CHUNK, zbody, 0)

        base_r = sid * rows_per_sub
        for k in range(rows_per_sub // CHUNK):
            pltpu.sync_copy(rows.at[0],
                            acc.at[pl.ds(base_r + k * CHUNK, CHUNK)])
        plsc.subcore_barrier()

        npairs = n_chunks // 2

        def body(i, carry):
            row0 = wid * n_chunks + i * 2
            pltpu.sync_copy(src_hbm.at[pl.ds(row0, 2)], sidx)
            pltpu.sync_copy(dst_hbm.at[pl.ds(row0, 2)], didx)
            cp0 = pltpu.async_copy(table_hbm.at[sidx.at[0]], rows.at[0], sem0)
            cp1 = pltpu.async_copy(table_hbm.at[sidx.at[1]], rows.at[1], sem1)
            cp0.wait()
            pltpu.sync_copy(rows.at[0], acc.at[didx.at[0]], add=True)
            cp1.wait()
            pltpu.sync_copy(rows.at[1], acc.at[didx.at[1]], add=True)
            return carry

        lax.fori_loop(0, npairs, body, 0)
        plsc.subcore_barrier()

        pltpu.sync_copy(acc.at[pl.ds(base_r, rows_per_sub)],
                        out_hbm.at[cid, pl.ds(base_r, rows_per_sub)])

    return seg


def _proj(x, W, b, bm):
    M, K = x.shape

    def body(x_ref, w_ref, b_ref, o_ref):
        o_ref[...] = jnp.dot(x_ref[...], w_ref[...],
                             preferred_element_type=jnp.float32) + b_ref[...]

    return pl.pallas_call(
        body,
        grid=(M // bm,),
        in_specs=[
            pl.BlockSpec((bm, K), lambda i: (i, 0)),
            pl.BlockSpec((K, H), lambda i: (0, 0)),
            pl.BlockSpec((1, H), lambda i: (0, 0)),
        ],
        out_specs=pl.BlockSpec((bm, H), lambda i: (i, 0)),
        out_shape=jax.ShapeDtypeStruct((M, H), jnp.float32),
    )(x, W, b.reshape(1, H))


def _proj_cell(x, W, b, deg_cs):
    M, K = x.shape
    bm = 128
    deg4 = deg_cs.reshape(NC, BIN_CELL // 128, 1, 128)

    def body(x_ref, w_ref, b_ref, c_ref, o_ref, os_ref):
        h = jnp.dot(x_ref[...], w_ref[...],
                    preferred_element_type=jnp.float32) + b_ref[...]
        c = c_ref[0, 0, 0] + c_ref[1, 0, 0]
        norm = lax.rsqrt(jnp.maximum(c, 1.0))
        o_ref[...] = h
        os_ref[...] = h * norm[:, None]

    return pl.pallas_call(
        body,
        grid=(M // bm,),
        in_specs=[
            pl.BlockSpec((bm, K), lambda i: (i, 0)),
            pl.BlockSpec((K, H), lambda i: (0, 0)),
            pl.BlockSpec((1, H), lambda i: (0, 0)),
            pl.BlockSpec((NC, 1, 1, 128), lambda i: (0, i, 0, 0)),
        ],
        out_specs=[
            pl.BlockSpec((bm, H), lambda i: (i, 0)),
            pl.BlockSpec((bm, H), lambda i: (i, 0)),
        ],
        out_shape=[
            jax.ShapeDtypeStruct((M, H), jnp.float32),
            jax.ShapeDtypeStruct((M, H), jnp.float32),
        ],
    )(x, W, b.reshape(1, H), deg4)


def _sage_update(h, sums, cnt, Ws, Wn, b):
    M = h.shape[0]
    bm = 128
    grid = (M + bm - 1) // bm
    cnt4 = cnt.reshape(NC, BIN_PROT // 128, 1, 128)

    def body(h_ref, s_ref, c_ref, ws_ref, wn_ref, b_ref, o_ref):
        s = s_ref[0] + s_ref[1]
        c = c_ref[0, 0, 0] + c_ref[1, 0, 0]
        inv = 1.0 / jnp.maximum(c, 1.0)
        neigh = s * inv[:, None]
        z = (jnp.dot(h_ref[...], ws_ref[...],
                     preferred_element_type=jnp.float32)
             + jnp.dot(neigh, wn_ref[...],
                       preferred_element_type=jnp.float32)
             + b_ref[...])
        o_ref[...] = h_ref[...] + jnp.maximum(z, 0.0)

    return pl.pallas_call(
        body,
        grid=(grid,),
        in_specs=[
            pl.BlockSpec((bm, H), lambda i: (i, 0)),
            pl.BlockSpec((NC, bm, H), lambda i: (0, i, 0)),
            pl.BlockSpec((NC, 1, 1, 128), lambda i: (0, i, 0, 0)),
            pl.BlockSpec((H, H), lambda i: (0, 0)),
            pl.BlockSpec((H, H), lambda i: (0, 0)),
            pl.BlockSpec((1, H), lambda i: (0, 0)),
        ],
        out_specs=pl.BlockSpec((bm, H), lambda i: (i, 0)),
        out_shape=jax.ShapeDtypeStruct((M, H), jnp.float32),
    )(h, sums, cnt4, Ws, Wn, b.reshape(1, H))


def _clp_update(h2, h0, sums, cnt, Wg, bg):
    M = h2.shape[0]
    bm = 128
    grid = (M + bm - 1) // bm
    cnt4 = cnt.reshape(NC, BIN_PROT // 128, 1, 128)

    def body(h2_ref, h0_ref, s_ref, c_ref, wg_ref, b_ref, o_ref):
        s = s_ref[0] + s_ref[1]
        c = c_ref[0, 0, 0] + c_ref[1, 0, 0]
        norm = lax.rsqrt(jnp.maximum(c, 1.0))
        agg = s * norm[:, None]
        z = jnp.dot(agg, wg_ref[...],
                    preferred_element_type=jnp.float32) + b_ref[...]
        o_ref[...] = h2_ref[...] + h0_ref[...] + jnp.maximum(z, 0.0)

    return pl.pallas_call(
        body,
        grid=(grid,),
        in_specs=[
            pl.BlockSpec((bm, H), lambda i: (i, 0)),
            pl.BlockSpec((bm, H), lambda i: (i, 0)),
            pl.BlockSpec((NC, bm, H), lambda i: (0, i, 0)),
            pl.BlockSpec((NC, 1, 1, 128), lambda i: (0, i, 0, 0)),
            pl.BlockSpec((H, H), lambda i: (0, 0)),
            pl.BlockSpec((1, H), lambda i: (0, 0)),
        ],
        out_specs=pl.BlockSpec((bm, H), lambda i: (i, 0)),
        out_shape=jax.ShapeDtypeStruct((M, H), jnp.float32),
    )(h2, h0, sums, cnt4, Wg, bg.reshape(1, H))


def kernel(drug_features, protein_features, cell_line_features,
           disease_features, pp_edge_index, clp_src, clp_dst,
           W_drug, b_drug, W_prot, b_prot, W_cell, b_cell, W_dis, b_dis,
           W_self1, W_neigh1, b_sage1, W_self2, W_neigh2, b_sage2,
           W_gcn, b_gcn):
    e_pp = pp_edge_index.shape[1]
    e_clp = clp_src.shape[0]
    per = NW * CHUNK * 2
    e_pp_pad = _pad_len(e_pp, per)
    e_clp_pad = _pad_len(e_clp, per)
    n_pp_chunks = e_pp_pad // (NW * CHUNK)
    n_clp_chunks = e_clp_pad // (NW * CHUNK)

    def pad_idx(a, n_pad, mode):
        if mode == "src":
            fill = jnp.arange(n_pad, dtype=jnp.int32) % 128
        elif mode == "dst_prot":
            fill = N_PROT + (jnp.arange(n_pad, dtype=jnp.int32) % 112)
        else:
            fill = N_CELL + (jnp.arange(n_pad, dtype=jnp.int32) % 128)
        return jnp.concatenate([a.astype(jnp.int32), fill])

    src_pp = pad_idx(pp_edge_index[0], e_pp_pad - e_pp, "src")
    dst_pp = pad_idx(pp_edge_index[1], e_pp_pad - e_pp, "dst_prot")
    clp_s_g = pad_idx(clp_src, e_clp_pad - e_clp, "src")
    clp_s_h = pad_idx(clp_src, e_clp_pad - e_clp, "dst_cell")
    clp_d = pad_idx(clp_dst, e_clp_pad - e_clp, "dst_prot")

    src_pp2 = src_pp.reshape(-1, CHUNK)
    dst_pp2 = dst_pp.reshape(-1, CHUNK)
    clp_s2 = clp_s_g.reshape(-1, CHUNK)
    clp_d2 = clp_d.reshape(-1, CHUNK)

    deg = _make_degrees(n_pp_chunks, n_clp_chunks)
    cnt_pp3, deg_cs3, deg_cd3 = deg(dst_pp, clp_s_h, clp_d)
    cnt_pp = cnt_pp3[:, :, 0]
    deg_cs = deg_cs3[:, :, 0]
    deg_cd = deg_cd3[:, :, 0]

    h_drug = _proj(drug_features, W_drug, b_drug, 256)
    h_prot0 = _proj(protein_features, W_prot, b_prot, 400)
    h_dis = _proj(disease_features, W_dis, b_dis, 256)
    h_cell, h_cell_scaled = _proj_cell(cell_line_features, W_cell, b_cell,
                                       deg_cs)

    segsum_pp = _make_segsum(n_pp_chunks, BIN_PROT)
    segsum_clp = _make_segsum(n_clp_chunks, BIN_PROT)

    sums1 = segsum_pp(src_pp2, dst_pp2, h_prot0)
    h1 = _sage_update(h_prot0, sums1, cnt_pp, W_self1, W_neigh1, b_sage1)
    sums2 = segsum_pp(src_pp2, dst_pp2, h1)
    h2 = _sage_update(h1, sums2, cnt_pp, W_self2, W_neigh2, b_sage2)

    aggc = segsum_clp(clp_s2, clp_d2, h_cell_scaled)
    h_prot = _clp_update(h2, h_prot0, aggc, deg_cd, W_gcn, b_gcn)

    return (h_drug, h_prot, h_cell, h_dis)


# speedup vs baseline: 5.0056x; 1.1636x over previous
import functools

import jax
import jax.numpy as jnp
from jax import lax
from jax.experimental import pallas as pl
from jax.experimental.pallas import tpu as pltpu
from jax.experimental.pallas import tpu_sc as plsc

N_PROT = 10000
N_CELL = 1024
H = 128

NC, NS = 2, 16
NW = NC * NS
CHUNK = 128

BIN_PROT = 10240
BIN_CELL = 2048


def _pad_len(e, per):
    return ((e + per - 1) // per) * per


def _make_degrees(n_pp_chunks, n_clp_chunks):
    mesh = plsc.VectorSubcoreMesh(
        core_axis_name="c", subcore_axis_name="s", num_cores=NC,
        num_subcores=NS)
    pp_per = BIN_PROT // NS
    cs_per = BIN_CELL // NS

    @functools.partial(
        pl.kernel,
        out_type=(
            jax.ShapeDtypeStruct((NC, BIN_PROT, H), jnp.float32),
            jax.ShapeDtypeStruct((NC, BIN_CELL, H), jnp.float32),
            jax.ShapeDtypeStruct((NC, BIN_PROT, H), jnp.float32),
        ),
        mesh=mesh,
        scratch_types=[
            pltpu.VMEM((CHUNK,), jnp.int32),
            pltpu.VMEM((CHUNK, H), jnp.float32),
            pltpu.VMEM((CHUNK, H), jnp.float32),
            pltpu.VMEM_SHARED((BIN_PROT, H), jnp.float32),
        ],
    )
    def deg(ppd_hbm, cls_hbm, cld_hbm, out_pp, out_cs, out_cd,
            idxb, onesb, zrows, acc):
        cid = lax.axis_index("c")
        sid = lax.axis_index("s")
        wid = cid * NS + sid

        ov = jnp.ones((16,), jnp.float32)
        zv = jnp.zeros((16,), jnp.float32)

        def fill(r, carry):
            for c in range(8):
                onesb[r, pl.ds(c * 16, 16)] = ov
                zrows[r, pl.ds(c * 16, 16)] = zv
            return carry

        lax.fori_loop(0, CHUNK, fill, 0)

        def zero_big():
            base_r = sid * pp_per
            for k in range(pp_per // CHUNK):
                pltpu.sync_copy(zrows,
                                acc.at[pl.ds(base_r + k * CHUNK, CHUNK)])

        zero_big()
        plsc.subcore_barrier()

        def run(idx_hbm, a, n_chunks):
            def body(j, carry):
                base = (wid * n_chunks + j) * CHUNK
                pltpu.sync_copy(idx_hbm.at[pl.ds(base, CHUNK)], idxb)
                pltpu.sync_copy(onesb, a.at[idxb], add=True)
                return carry
            lax.fori_loop(0, n_chunks, body, 0)

        base_r = sid * pp_per

        run(ppd_hbm, acc, n_pp_chunks)
        plsc.subcore_barrier()
        pltpu.sync_copy(acc.at[pl.ds(base_r, pp_per)],
                        out_pp.at[cid, pl.ds(base_r, pp_per)])
        plsc.subcore_barrier()

        zero_big()
        plsc.subcore_barrier()
        run(cld_hbm, acc, n_clp_chunks)
        plsc.subcore_barrier()
        pltpu.sync_copy(acc.at[pl.ds(base_r, pp_per)],
                        out_cd.at[cid, pl.ds(base_r, pp_per)])
        plsc.subcore_barrier()

        zero_big()
        plsc.subcore_barrier()
        run(cls_hbm, acc, n_clp_chunks)
        plsc.subcore_barrier()
        pltpu.sync_copy(acc.at[pl.ds(sid * cs_per, cs_per)],
                        out_cs.at[cid, pl.ds(sid * cs_per, cs_per)])

    return deg


def _make_segsum(n_chunks, n_bins):
    mesh = plsc.VectorSubcoreMesh(
        core_axis_name="c", subcore_axis_name="s", num_cores=NC,
        num_subcores=NS)
    rows_per_sub = n_bins // NS
    NBUF = 2
    NB = 8
    assert n_chunks % NB == 0

    @functools.partial(
        pl.kernel,
        out_type=jax.ShapeDtypeStruct((NC, n_bins, H), jnp.float32),
        mesh=mesh,
        scratch_types=[
            pltpu.VMEM((NB, CHUNK), jnp.int32),
            pltpu.VMEM((NB, CHUNK), jnp.int32),
            pltpu.VMEM((NBUF, CHUNK, H), jnp.float32),
            pltpu.VMEM_SHARED((n_bins, H), jnp.float32),
        ] + [pltpu.SemaphoreType.DMA] * NBUF,
    )
    def seg(src_hbm, dst_hbm, table_hbm, out_hbm,
            sidx, didx, rows, acc, *sems):
        cid = lax.axis_index("c")
        sid = lax.axis_index("s")
        wid = cid * NS + sid

        zv = jnp.zeros((16,), jnp.float32)

        def zbody(r, carry):
            for c in range(8):
                rows[0, r, pl.ds(c * 16, 16)] = zv
            return carry

        lax.fori_loop(0, CHUNK, zbody, 0)

        base_r = sid * rows_per_sub
        for k in range(rows_per_sub // CHUNK):
            pltpu.sync_copy(rows.at[0],
                            acc.at[pl.ds(base_r + k * CHUNK, CHUNK)])
        plsc.subcore_barrier()

        def body(ib, carry):
            row0 = wid * n_chunks + ib * NB
            pltpu.sync_copy(src_hbm.at[pl.ds(row0, NB)], sidx)
            pltpu.sync_copy(dst_hbm.at[pl.ds(row0, NB)], didx)
            copies = [None] * NB
            for j in range(min(NBUF, NB)):
                copies[j] = pltpu.async_copy(
                    table_hbm.at[sidx.at[j]], rows.at[j % NBUF], sems[j % NBUF])
            for j in range(NB):
                copies[j].wait()
                pltpu.sync_copy(rows.at[j % NBUF], acc.at[didx.at[j]],
                                add=True)
                nxt = j + NBUF
                if nxt < NB:
                    copies[nxt] = pltpu.async_copy(
                        table_hbm.at[sidx.at[nxt]], rows.at[nxt % NBUF],
                        sems[nxt % NBUF])
            return carry

        lax.fori_loop(0, n_chunks // NB, body, 0)
        plsc.subcore_barrier()

        pltpu.sync_copy(acc.at[pl.ds(base_r, rows_per_sub)],
                        out_hbm.at[cid, pl.ds(base_r, rows_per_sub)])

    return seg


def _proj(x, W, b, bm):
    M, K = x.shape

    def body(x_ref, w_ref, b_ref, o_ref):
        o_ref[...] = jnp.dot(x_ref[...], w_ref[...],
                             preferred_element_type=jnp.float32) + b_ref[...]

    return pl.pallas_call(
        body,
        grid=(M // bm,),
        in_specs=[
            pl.BlockSpec((bm, K), lambda i: (i, 0)),
            pl.BlockSpec((K, H), lambda i: (0, 0)),
            pl.BlockSpec((1, H), lambda i: (0, 0)),
        ],
        out_specs=pl.BlockSpec((bm, H), lambda i: (i, 0)),
        out_shape=jax.ShapeDtypeStruct((M, H), jnp.float32),
    )(x, W, b.reshape(1, H))


def _proj_cell(x, W, b, deg_cs):
    M, K = x.shape
    bm = 128
    deg4 = deg_cs.reshape(NC, BIN_CELL // 128, 1, 128)

    def body(x_ref, w_ref, b_ref, c_ref, o_ref, os_ref):
        h = jnp.dot(x_ref[...], w_ref[...],
                    preferred_element_type=jnp.float32) + b_ref[...]
        c = c_ref[0, 0, 0] + c_ref[1, 0, 0]
        norm = lax.rsqrt(jnp.maximum(c, 1.0))
        o_ref[...] = h
        os_ref[...] = h * norm[:, None]

    return pl.pallas_call(
        body,
        grid=(M // bm,),
        in_specs=[
            pl.BlockSpec((bm, K), lambda i: (i, 0)),
            pl.BlockSpec((K, H), lambda i: (0, 0)),
            pl.BlockSpec((1, H), lambda i: (0, 0)),
            pl.BlockSpec((NC, 1, 1, 128), lambda i: (0, i, 0, 0)),
        ],
        out_specs=[
            pl.BlockSpec((bm, H), lambda i: (i, 0)),
            pl.BlockSpec((bm, H), lambda i: (i, 0)),
        ],
        out_shape=[
            jax.ShapeDtypeStruct((M, H), jnp.float32),
            jax.ShapeDtypeStruct((M, H), jnp.float32),
        ],
    )(x, W, b.reshape(1, H), deg4)


def _sage_update(h, sums, cnt, Ws, Wn, b):
    M = h.shape[0]
    bm = 128
    grid = (M + bm - 1) // bm
    cnt4 = cnt.reshape(NC, BIN_PROT // 128, 1, 128)

    def body(h_ref, s_ref, c_ref, ws_ref, wn_ref, b_ref, o_ref):
        s = s_ref[0] + s_ref[1]
        c = c_ref[0, 0, 0] + c_ref[1, 0, 0]
        inv = 1.0 / jnp.maximum(c, 1.0)
        neigh = s * inv[:, None]
        z = (jnp.dot(h_ref[...], ws_ref[...],
                     preferred_element_type=jnp.float32)
             + jnp.dot(neigh, wn_ref[...],
                       preferred_element_type=jnp.float32)
             + b_ref[...])
        o_ref[...] = h_ref[...] + jnp.maximum(z, 0.0)

    return pl.pallas_call(
        body,
        grid=(grid,),
        in_specs=[
            pl.BlockSpec((bm, H), lambda i: (i, 0)),
            pl.BlockSpec((NC, bm, H), lambda i: (0, i, 0)),
            pl.BlockSpec((NC, 1, 1, 128), lambda i: (0, i, 0, 0)),
            pl.BlockSpec((H, H), lambda i: (0, 0)),
            pl.BlockSpec((H, H), lambda i: (0, 0)),
            pl.BlockSpec((1, H), lambda i: (0, 0)),
        ],
        out_specs=pl.BlockSpec((bm, H), lambda i: (i, 0)),
        out_shape=jax.ShapeDtypeStruct((M, H), jnp.float32),
    )(h, sums, cnt4, Ws, Wn, b.reshape(1, H))


def _clp_update(h2, h0, sums, cnt, Wg, bg):
    M = h2.shape[0]
    bm = 128
    grid = (M + bm - 1) // bm
    cnt4 = cnt.reshape(NC, BIN_PROT // 128, 1, 128)

    def body(h2_ref, h0_ref, s_ref, c_ref, wg_ref, b_ref, o_ref):
        s = s_ref[0] + s_ref[1]
        c = c_ref[0, 0, 0] + c_ref[1, 0, 0]
        norm = lax.rsqrt(jnp.maximum(c, 1.0))
        agg = s * norm[:, None]
        z = jnp.dot(agg, wg_ref[...],
                    preferred_element_type=jnp.float32) + b_ref[...]
        o_ref[...] = h2_ref[...] + h0_ref[...] + jnp.maximum(z, 0.0)

    return pl.pallas_call(
        body,
        grid=(grid,),
        in_specs=[
            pl.BlockSpec((bm, H), lambda i: (i, 0)),
            pl.BlockSpec((bm, H), lambda i: (i, 0)),
            pl.BlockSpec((NC, bm, H), lambda i: (0, i, 0)),
            pl.BlockSpec((NC, 1, 1, 128), lambda i: (0, i, 0, 0)),
            pl.BlockSpec((H, H), lambda i: (0, 0)),
            pl.BlockSpec((1, H), lambda i: (0, 0)),
        ],
        out_specs=pl.BlockSpec((bm, H), lambda i: (i, 0)),
        out_shape=jax.ShapeDtypeStruct((M, H), jnp.float32),
    )(h2, h0, sums, cnt4, Wg, bg.reshape(1, H))


def kernel(drug_features, protein_features, cell_line_features,
           disease_features, pp_edge_index, clp_src, clp_dst,
           W_drug, b_drug, W_prot, b_prot, W_cell, b_cell, W_dis, b_dis,
           W_self1, W_neigh1, b_sage1, W_self2, W_neigh2, b_sage2,
           W_gcn, b_gcn):
    e_pp = pp_edge_index.shape[1]
    e_clp = clp_src.shape[0]
    per = NW * CHUNK * 8
    e_pp_pad = _pad_len(e_pp, per)
    e_clp_pad = _pad_len(e_clp, per)
    n_pp_chunks = e_pp_pad // (NW * CHUNK)
    n_clp_chunks = e_clp_pad // (NW * CHUNK)

    def pad_idx(a, n_pad, mode):
        if mode == "src":
            fill = jnp.arange(n_pad, dtype=jnp.int32) % 128
        elif mode == "dst_prot":
            fill = N_PROT + (jnp.arange(n_pad, dtype=jnp.int32) % 112)
        else:
            fill = N_CELL + (jnp.arange(n_pad, dtype=jnp.int32) % 128)
        return jnp.concatenate([a.astype(jnp.int32), fill])

    src_pp = pad_idx(pp_edge_index[0], e_pp_pad - e_pp, "src")
    dst_pp = pad_idx(pp_edge_index[1], e_pp_pad - e_pp, "dst_prot")
    clp_s_g = pad_idx(clp_src, e_clp_pad - e_clp, "src")
    clp_s_h = pad_idx(clp_src, e_clp_pad - e_clp, "dst_cell")
    clp_d = pad_idx(clp_dst, e_clp_pad - e_clp, "dst_prot")

    src_pp2 = src_pp.reshape(-1, CHUNK)
    dst_pp2 = dst_pp.reshape(-1, CHUNK)
    clp_s2 = clp_s_g.reshape(-1, CHUNK)
    clp_d2 = clp_d.reshape(-1, CHUNK)

    deg = _make_degrees(n_pp_chunks, n_clp_chunks)
    cnt_pp3, deg_cs3, deg_cd3 = deg(dst_pp, clp_s_h, clp_d)
    cnt_pp = cnt_pp3[:, :, 0]
    deg_cs = deg_cs3[:, :, 0]
    deg_cd = deg_cd3[:, :, 0]

    h_drug = _proj(drug_features, W_drug, b_drug, 256)
    h_prot0 = _proj(protein_features, W_prot, b_prot, 400)
    h_dis = _proj(disease_features, W_dis, b_dis, 256)
    h_cell, h_cell_scaled = _proj_cell(cell_line_features, W_cell, b_cell,
                                       deg_cs)

    segsum_pp = _make_segsum(n_pp_chunks, BIN_PROT)
    segsum_clp = _make_segsum(n_clp_chunks, BIN_PROT)

    sums1 = segsum_pp(src_pp2, dst_pp2, h_prot0)
    h1 = _sage_update(h_prot0, sums1, cnt_pp, W_self1, W_neigh1, b_sage1)
    sums2 = segsum_pp(src_pp2, dst_pp2, h1)
    h2 = _sage_update(h1, sums2, cnt_pp, W_self2, W_neigh2, b_sage2)

    aggc = segsum_clp(clp_s2, clp_d2, h_cell_scaled)
    h_prot = _clp_update(h2, h_prot0, aggc, deg_cd, W_gcn, b_gcn)

    return (h_drug, h_prot, h_cell, h_dis)


# speedup vs baseline: 5.5744x; 1.1136x over previous
import functools

import jax
import jax.numpy as jnp
from jax import lax
from jax.experimental import pallas as pl
from jax.experimental.pallas import tpu as pltpu
from jax.experimental.pallas import tpu_sc as plsc

N_PROT = 10000
N_CELL = 1024
H = 128

NC, NS = 2, 16
NW = NC * NS
CHUNK = 128

BIN_PROT = 10240
BIN_CELL = 2048


def _pad_len(e, per):
    return ((e + per - 1) // per) * per


def _make_degrees(n_pp_chunks, n_clp_chunks):
    mesh = plsc.VectorSubcoreMesh(
        core_axis_name="c", subcore_axis_name="s", num_cores=NC,
        num_subcores=NS)
    pp_per = BIN_PROT // NS
    cs_per = BIN_CELL // NS
    NBINS = BIN_PROT + BIN_CELL
    NB = 8

    @functools.partial(
        pl.kernel,
        out_type=(
            jax.ShapeDtypeStruct((NC, BIN_PROT, H), jnp.float32),
            jax.ShapeDtypeStruct((NC, BIN_CELL, H), jnp.float32),
            jax.ShapeDtypeStruct((NC, BIN_PROT, H), jnp.float32),
        ),
        mesh=mesh,
        scratch_types=[
            pltpu.VMEM((NB, CHUNK), jnp.int32),
            pltpu.VMEM((CHUNK, H), jnp.float32),
            pltpu.VMEM((64, H), jnp.float32),
            pltpu.VMEM_SHARED((NBINS, H), jnp.float32),
        ],
    )
    def deg(ppd_hbm, cls_hbm, cld_hbm, out_pp, out_cs, out_cd,
            idxb, onesb, zrows, acc):
        cid = lax.axis_index("c")
        sid = lax.axis_index("s")
        wid = cid * NS + sid

        ov = jnp.ones((16,), jnp.float32)
        zv = jnp.zeros((16,), jnp.float32)

        def fill(r, carry):
            for c in range(8):
                onesb[r, pl.ds(c * 16, 16)] = ov
            return carry

        def fillz(r, carry):
            for c in range(8):
                zrows[r, pl.ds(c * 16, 16)] = zv
            return carry

        lax.fori_loop(0, CHUNK, fill, 0)
        lax.fori_loop(0, 64, fillz, 0)

        def zero(nrows):
            base_r = sid * (nrows // NS)
            for k in range((nrows // NS) // 64):
                pltpu.sync_copy(zrows,
                                acc.at[pl.ds(base_r + k * 64, 64)])

        def run(idx2_hbm, n_chunks):
            def body(ib, carry):
                row0 = wid * n_chunks + ib * NB
                pltpu.sync_copy(idx2_hbm.at[pl.ds(row0, NB)], idxb)
                for j in range(NB):
                    pltpu.sync_copy(onesb, acc.at[idxb.at[j]], add=True)
                return carry
            lax.fori_loop(0, n_chunks // NB, body, 0)

        zero(NBINS)
        plsc.subcore_barrier()
        run(ppd_hbm, n_pp_chunks)
        run(cls_hbm, n_clp_chunks)
        plsc.subcore_barrier()
        pltpu.sync_copy(acc.at[pl.ds(sid * pp_per, pp_per)],
                        out_pp.at[cid, pl.ds(sid * pp_per, pp_per)])
        pltpu.sync_copy(acc.at[pl.ds(BIN_PROT + sid * cs_per, cs_per)],
                        out_cs.at[cid, pl.ds(sid * cs_per, cs_per)])
        plsc.subcore_barrier()

        zero(BIN_PROT)
        plsc.subcore_barrier()
        run(cld_hbm, n_clp_chunks)
        plsc.subcore_barrier()
        pltpu.sync_copy(acc.at[pl.ds(sid * pp_per, pp_per)],
                        out_cd.at[cid, pl.ds(sid * pp_per, pp_per)])

    return deg


def _make_segsum(n_chunks, n_bins):
    mesh = plsc.VectorSubcoreMesh(
        core_axis_name="c", subcore_axis_name="s", num_cores=NC,
        num_subcores=NS)
    rows_per_sub = n_bins // NS
    NBUF = 2
    NB = 8
    assert n_chunks % NB == 0

    @functools.partial(
        pl.kernel,
        out_type=jax.ShapeDtypeStruct((NC, n_bins, H), jnp.float32),
        mesh=mesh,
        scratch_types=[
            pltpu.VMEM((NB, CHUNK), jnp.int32),
            pltpu.VMEM((NB, CHUNK), jnp.int32),
            pltpu.VMEM((NBUF, CHUNK, H), jnp.float32),
            pltpu.VMEM_SHARED((n_bins, H), jnp.float32),
        ] + [pltpu.SemaphoreType.DMA] * NBUF,
    )
    def seg(src_hbm, dst_hbm, table_hbm, out_hbm,
            sidx, didx, rows, acc, *sems):
        cid = lax.axis_index("c")
        sid = lax.axis_index("s")
        wid = cid * NS + sid

        zv = jnp.zeros((16,), jnp.float32)

        def zbody(r, carry):
            for c in range(8):
                rows[0, r, pl.ds(c * 16, 16)] = zv
            return carry

        lax.fori_loop(0, CHUNK, zbody, 0)

        base_r = sid * rows_per_sub
        for k in range(rows_per_sub // CHUNK):
            pltpu.sync_copy(rows.at[0],
                            acc.at[pl.ds(base_r + k * CHUNK, CHUNK)])
        plsc.subcore_barrier()

        def body(ib, carry):
            row0 = wid * n_chunks + ib * NB
            pltpu.sync_copy(src_hbm.at[pl.ds(row0, NB)], sidx)
            pltpu.sync_copy(dst_hbm.at[pl.ds(row0, NB)], didx)
            copies = [None] * NB
            for j in range(min(NBUF, NB)):
                copies[j] = pltpu.async_copy(
                    table_hbm.at[sidx.at[j]], rows.at[j % NBUF], sems[j % NBUF])
            for j in range(NB):
                copies[j].wait()
                pltpu.sync_copy(rows.at[j % NBUF], acc.at[didx.at[j]],
                                add=True)
                nxt = j + NBUF
                if nxt < NB:
                    copies[nxt] = pltpu.async_copy(
                        table_hbm.at[sidx.at[nxt]], rows.at[nxt % NBUF],
                        sems[nxt % NBUF])
            return carry

        lax.fori_loop(0, n_chunks // NB, body, 0)
        plsc.subcore_barrier()

        pltpu.sync_copy(acc.at[pl.ds(base_r, rows_per_sub)],
                        out_hbm.at[cid, pl.ds(base_r, rows_per_sub)])

    return seg


def _proj(x, W, b, bm):
    M, K = x.shape

    def body(x_ref, w_ref, b_ref, o_ref):
        o_ref[...] = jnp.dot(x_ref[...], w_ref[...],
                             preferred_element_type=jnp.float32) + b_ref[...]

    return pl.pallas_call(
        body,
        grid=(M // bm,),
        in_specs=[
            pl.BlockSpec((bm, K), lambda i: (i, 0)),
            pl.BlockSpec((K, H), lambda i: (0, 0)),
            pl.BlockSpec((1, H), lambda i: (0, 0)),
        ],
        out_specs=pl.BlockSpec((bm, H), lambda i: (i, 0)),
        out_shape=jax.ShapeDtypeStruct((M, H), jnp.float32),
    )(x, W, b.reshape(1, H))


def _proj_cell(x, W, b, deg_cs):
    M, K = x.shape
    bm = 128
    deg4 = deg_cs.reshape(NC, BIN_CELL // 128, 1, 128)

    def body(x_ref, w_ref, b_ref, c_ref, o_ref, os_ref):
        h = jnp.dot(x_ref[...], w_ref[...],
                    preferred_element_type=jnp.float32) + b_ref[...]
        c = c_ref[0, 0, 0] + c_ref[1, 0, 0]
        norm = lax.rsqrt(jnp.maximum(c, 1.0))
        o_ref[...] = h
        os_ref[...] = h * norm[:, None]

    return pl.pallas_call(
        body,
        grid=(M // bm,),
        in_specs=[
            pl.BlockSpec((bm, K), lambda i: (i, 0)),
            pl.BlockSpec((K, H), lambda i: (0, 0)),
            pl.BlockSpec((1, H), lambda i: (0, 0)),
            pl.BlockSpec((NC, 1, 1, 128), lambda i: (0, i, 0, 0)),
        ],
        out_specs=[
            pl.BlockSpec((bm, H), lambda i: (i, 0)),
            pl.BlockSpec((bm, H), lambda i: (i, 0)),
        ],
        out_shape=[
            jax.ShapeDtypeStruct((M, H), jnp.float32),
            jax.ShapeDtypeStruct((M, H), jnp.float32),
        ],
    )(x, W, b.reshape(1, H), deg4)


def _sage_update(h, sums, cnt, Ws, Wn, b):
    M = h.shape[0]
    bm = 128
    grid = (M + bm - 1) // bm
    cnt4 = cnt.reshape(NC, BIN_PROT // 128, 1, 128)

    def body(h_ref, s_ref, c_ref, ws_ref, wn_ref, b_ref, o_ref):
        s = s_ref[0] + s_ref[1]
        c = c_ref[0, 0, 0] + c_ref[1, 0, 0]
        inv = 1.0 / jnp.maximum(c, 1.0)
        neigh = s * inv[:, None]
        z = (jnp.dot(h_ref[...], ws_ref[...],
                     preferred_element_type=jnp.float32)
             + jnp.dot(neigh, wn_ref[...],
                       preferred_element_type=jnp.float32)
             + b_ref[...])
        o_ref[...] = h_ref[...] + jnp.maximum(z, 0.0)

    return pl.pallas_call(
        body,
        grid=(grid,),
        in_specs=[
            pl.BlockSpec((bm, H), lambda i: (i, 0)),
            pl.BlockSpec((NC, bm, H), lambda i: (0, i, 0)),
            pl.BlockSpec((NC, 1, 1, 128), lambda i: (0, i, 0, 0)),
            pl.BlockSpec((H, H), lambda i: (0, 0)),
            pl.BlockSpec((H, H), lambda i: (0, 0)),
            pl.BlockSpec((1, H), lambda i: (0, 0)),
        ],
        out_specs=pl.BlockSpec((bm, H), lambda i: (i, 0)),
        out_shape=jax.ShapeDtypeStruct((M, H), jnp.float32),
    )(h, sums, cnt4, Ws, Wn, b.reshape(1, H))


def _clp_update(h2, h0, sums, cnt, Wg, bg):
    M = h2.shape[0]
    bm = 128
    grid = (M + bm - 1) // bm
    cnt4 = cnt.reshape(NC, BIN_PROT // 128, 1, 128)

    def body(h2_ref, h0_ref, s_ref, c_ref, wg_ref, b_ref, o_ref):
        s = s_ref[0] + s_ref[1]
        c = c_ref[0, 0, 0] + c_ref[1, 0, 0]
        norm = lax.rsqrt(jnp.maximum(c, 1.0))
        agg = s * norm[:, None]
        z = jnp.dot(agg, wg_ref[...],
                    preferred_element_type=jnp.float32) + b_ref[...]
        o_ref[...] = h2_ref[...] + h0_ref[...] + jnp.maximum(z, 0.0)

    return pl.pallas_call(
        body,
        grid=(grid,),
        in_specs=[
            pl.BlockSpec((bm, H), lambda i: (i, 0)),
            pl.BlockSpec((bm, H), lambda i: (i, 0)),
            pl.BlockSpec((NC, bm, H), lambda i: (0, i, 0)),
            pl.BlockSpec((NC, 1, 1, 128), lambda i: (0, i, 0, 0)),
            pl.BlockSpec((H, H), lambda i: (0, 0)),
            pl.BlockSpec((1, H), lambda i: (0, 0)),
        ],
        out_specs=pl.BlockSpec((bm, H), lambda i: (i, 0)),
        out_shape=jax.ShapeDtypeStruct((M, H), jnp.float32),
    )(h2, h0, sums, cnt4, Wg, bg.reshape(1, H))


def kernel(drug_features, protein_features, cell_line_features,
           disease_features, pp_edge_index, clp_src, clp_dst,
           W_drug, b_drug, W_prot, b_prot, W_cell, b_cell, W_dis, b_dis,
           W_self1, W_neigh1, b_sage1, W_self2, W_neigh2, b_sage2,
           W_gcn, b_gcn):
    e_pp = pp_edge_index.shape[1]
    e_clp = clp_src.shape[0]
    per = NW * CHUNK * 8
    e_pp_pad = _pad_len(e_pp, per)
    e_clp_pad = _pad_len(e_clp, per)
    n_pp_chunks = e_pp_pad // (NW * CHUNK)
    n_clp_chunks = e_clp_pad // (NW * CHUNK)

    def pad_idx(a, n_pad, mode):
        if mode == "src":
            fill = jnp.arange(n_pad, dtype=jnp.int32) % 128
        elif mode == "dst_prot":
            fill = N_PROT + (jnp.arange(n_pad, dtype=jnp.int32) % 112)
        else:
            fill = N_CELL + (jnp.arange(n_pad, dtype=jnp.int32) % 128)
        return jnp.concatenate([a.astype(jnp.int32), fill])

    src_pp = pad_idx(pp_edge_index[0], e_pp_pad - e_pp, "src")
    dst_pp = pad_idx(pp_edge_index[1], e_pp_pad - e_pp, "dst_prot")
    clp_s_g = pad_idx(clp_src, e_clp_pad - e_clp, "src")
    clp_s_h = BIN_PROT + pad_idx(clp_src, e_clp_pad - e_clp, "dst_cell")
    clp_d = pad_idx(clp_dst, e_clp_pad - e_clp, "dst_prot")

    src_pp2 = src_pp.reshape(-1, CHUNK)
    dst_pp2 = dst_pp.reshape(-1, CHUNK)
    clp_s2 = clp_s_g.reshape(-1, CHUNK)
    clp_d2 = clp_d.reshape(-1, CHUNK)

    deg = _make_degrees(n_pp_chunks, n_clp_chunks)
    cnt_pp3, deg_cs3, deg_cd3 = deg(dst_pp2, clp_s_h.reshape(-1, CHUNK), clp_d2)
    cnt_pp = cnt_pp3[:, :, 0]
    deg_cs = deg_cs3[:, :, 0]
    deg_cd = deg_cd3[:, :, 0]

    h_drug = _proj(drug_features, W_drug, b_drug, 256)
    h_prot0 = _proj(protein_features, W_prot, b_prot, 400)
    h_dis = _proj(disease_features, W_dis, b_dis, 256)
    h_cell, h_cell_scaled = _proj_cell(cell_line_features, W_cell, b_cell,
                                       deg_cs)

    segsum_pp = _make_segsum(n_pp_chunks, BIN_PROT)
    segsum_clp = _make_segsum(n_clp_chunks, BIN_PROT)

    sums1 = segsum_pp(src_pp2, dst_pp2, h_prot0)
    h1 = _sage_update(h_prot0, sums1, cnt_pp, W_self1, W_neigh1, b_sage1)
    sums2 = segsum_pp(src_pp2, dst_pp2, h1)
    h2 = _sage_update(h1, sums2, cnt_pp, W_self2, W_neigh2, b_sage2)

    aggc = segsum_clp(clp_s2, clp_d2, h_cell_scaled)
    h_prot = _clp_update(h2, h_prot0, aggc, deg_cd, W_gcn, b_gcn)

    return (h_drug, h_prot, h_cell, h_dis)
